# baseline (device time: 10668 ns/iter reference)
import jax
import jax.numpy as jnp
from jax import lax
from jax.experimental import pallas as pl
from jax.experimental.pallas import tpu as pltpu

N_DEV = 16
GLOBAL_ROWS = 8192
G = 4


def kernel(x):
    m_per, n = x.shape
    inv = 1.0 / GLOBAL_ROWS

    def body(
        x_ref,
        out_ref,
        plane_buf,
        col_buf,
        send_sems1,
        send_sems2,
        recv_sems1,
        recv_sems2,
    ):
        p = lax.axis_index("i")
        z = p // G
        k = p % G

        barrier = pltpu.get_barrier_semaphore()
        for kk in range(G):
            @pl.when(kk != k)
            def _(kk=kk):
                pl.semaphore_signal(
                    barrier, inc=16,
                    device_id=(G * z + kk,), device_id_type=pl.DeviceIdType.MESH,
                )
        for zz in range(G):
            @pl.when(zz != z)
            def _(zz=zz):
                pl.semaphore_signal(
                    barrier, inc=1,
                    device_id=(G * zz + k,), device_id_type=pl.DeviceIdType.MESH,
                )

        plane_buf[pl.ds(k, 1), :] = (
            jnp.sum(x_ref[:, :], axis=0, keepdims=True) * inv
        )

        pl.semaphore_wait(barrier, 48)

        for kk in range(G):
            @pl.when(kk != k)
            def _(kk=kk):
                rdma = pltpu.make_async_remote_copy(
                    src_ref=plane_buf.at[pl.ds(k, 1), :],
                    dst_ref=plane_buf.at[pl.ds(k, 1), :],
                    send_sem=send_sems1.at[kk],
                    recv_sem=recv_sems1.at[k],
                    device_id=(G * z + kk,),
                    device_id_type=pl.DeviceIdType.MESH,
                )
                rdma.start()
        for kk in range(G):
            @pl.when(kk != k)
            def _(kk=kk):
                recv = pltpu.make_async_remote_copy(
                    src_ref=plane_buf.at[pl.ds(kk, 1), :],
                    dst_ref=plane_buf.at[pl.ds(kk, 1), :],
                    send_sem=send_sems1.at[kk],
                    recv_sem=recv_sems1.at[kk],
                    device_id=(0,),
                    device_id_type=pl.DeviceIdType.MESH,
                )
                recv.wait_recv()

        col_buf[pl.ds(z, 1), :] = jnp.sum(plane_buf[:, :], axis=0, keepdims=True)

        pl.semaphore_wait(barrier, 3)

        for zz in range(G):
            @pl.when(zz != z)
            def _(zz=zz):
                rdma = pltpu.make_async_remote_copy(
                    src_ref=col_buf.at[pl.ds(z, 1), :],
                    dst_ref=col_buf.at[pl.ds(z, 1), :],
                    send_sem=send_sems2.at[zz],
                    recv_sem=recv_sems2.at[z],
                    device_id=(G * zz + k,),
                    device_id_type=pl.DeviceIdType.MESH,
                )
                rdma.start()
        for zz in range(G):
            @pl.when(zz != z)
            def _(zz=zz):
                recv = pltpu.make_async_remote_copy(
                    src_ref=col_buf.at[pl.ds(zz, 1), :],
                    dst_ref=col_buf.at[pl.ds(zz, 1), :],
                    send_sem=send_sems2.at[zz],
                    recv_sem=recv_sems2.at[zz],
                    device_id=(0,),
                    device_id_type=pl.DeviceIdType.MESH,
                )
                recv.wait_recv()

        out_ref[:, :] = jnp.sum(col_buf[:, :], axis=0, keepdims=True)

        for kk in range(G):
            @pl.when(kk != k)
            def _(kk=kk):
                snd = pltpu.make_async_remote_copy(
                    src_ref=plane_buf.at[pl.ds(k, 1), :],
                    dst_ref=plane_buf.at[pl.ds(k, 1), :],
                    send_sem=send_sems1.at[kk],
                    recv_sem=recv_sems1.at[kk],
                    device_id=(0,),
                    device_id_type=pl.DeviceIdType.MESH,
                )
                snd.wait_send()
        for zz in range(G):
            @pl.when(zz != z)
            def _(zz=zz):
                snd = pltpu.make_async_remote_copy(
                    src_ref=col_buf.at[pl.ds(z, 1), :],
                    dst_ref=col_buf.at[pl.ds(z, 1), :],
                    send_sem=send_sems2.at[zz],
                    recv_sem=recv_sems2.at[zz],
                    device_id=(0,),
                    device_id_type=pl.DeviceIdType.MESH,
                )
                snd.wait_send()

    return pl.pallas_call(
        body,
        out_shape=jax.ShapeDtypeStruct((1, n), jnp.float32),
        in_specs=[pl.BlockSpec(memory_space=pltpu.VMEM)],
        out_specs=pl.BlockSpec(memory_space=pltpu.VMEM),
        scratch_shapes=[
            pltpu.VMEM((G, n), jnp.float32),
            pltpu.VMEM((G, n), jnp.float32),
            pltpu.SemaphoreType.DMA((G,)),
            pltpu.SemaphoreType.DMA((G,)),
            pltpu.SemaphoreType.DMA((G,)),
            pltpu.SemaphoreType.DMA((G,)),
        ],
        compiler_params=pltpu.CompilerParams(collective_id=0),
    )(x)


# device time: 9753 ns/iter; 1.0938x vs baseline; 1.0938x over previous
import jax
import jax.numpy as jnp
from jax import lax
from jax.experimental import pallas as pl
from jax.experimental.pallas import tpu as pltpu

N_DEV = 16
GLOBAL_ROWS = 8192


def kernel(x):
    m_per, n = x.shape
    inv = 1.0 / GLOBAL_ROWS

    def body(x_hbm, out_ref, x_vmem, gather_ref, copy_sem, send_sems, recv_sems):
        my_pos = lax.axis_index("i")

        barrier = pltpu.get_barrier_semaphore()
        for d in range(1, N_DEV):
            pl.semaphore_signal(
                barrier, inc=1,
                device_id=((my_pos + d) % N_DEV,),
                device_id_type=pl.DeviceIdType.MESH,
            )

        cp = pltpu.make_async_copy(x_hbm, x_vmem, copy_sem)
        cp.start()
        cp.wait()
        gather_ref[pl.ds(my_pos, 1), :] = (
            jnp.sum(x_vmem[:, :], axis=0, keepdims=True) * inv
        )

        pl.semaphore_wait(barrier, N_DEV - 1)

        for d in range(1, N_DEV):
            rdma = pltpu.make_async_remote_copy(
                src_ref=gather_ref.at[pl.ds(my_pos, 1), :],
                dst_ref=gather_ref.at[pl.ds(my_pos, 1), :],
                send_sem=send_sems.at[(my_pos + d) % N_DEV],
                recv_sem=recv_sems.at[my_pos],
                device_id=((my_pos + d) % N_DEV,),
                device_id_type=pl.DeviceIdType.MESH,
            )
            rdma.start()

        for d in range(1, N_DEV):
            s = (my_pos + d) % N_DEV
            recv = pltpu.make_async_remote_copy(
                src_ref=gather_ref.at[pl.ds(s, 1), :],
                dst_ref=gather_ref.at[pl.ds(s, 1), :],
                send_sem=send_sems.at[s],
                recv_sem=recv_sems.at[s],
                device_id=(0,),
                device_id_type=pl.DeviceIdType.MESH,
            )
            recv.wait_recv()

        out_ref[:, :] = jnp.sum(gather_ref[:, :], axis=0, keepdims=True)

        for d in range(1, N_DEV):
            q = (my_pos + d) % N_DEV
            snd = pltpu.make_async_remote_copy(
                src_ref=gather_ref.at[pl.ds(my_pos, 1), :],
                dst_ref=gather_ref.at[pl.ds(my_pos, 1), :],
                send_sem=send_sems.at[q],
                recv_sem=recv_sems.at[q],
                device_id=(0,),
                device_id_type=pl.DeviceIdType.MESH,
            )
            snd.wait_send()

    return pl.pallas_call(
        body,
        out_shape=jax.ShapeDtypeStruct((1, n), jnp.float32),
        in_specs=[pl.BlockSpec(memory_space=pl.ANY)],
        out_specs=pl.BlockSpec(memory_space=pltpu.VMEM),
        scratch_shapes=[
            pltpu.VMEM((m_per, n), jnp.float32),
            pltpu.VMEM((N_DEV, n), jnp.float32),
            pltpu.SemaphoreType.DMA,
            pltpu.SemaphoreType.DMA((N_DEV,)),
            pltpu.SemaphoreType.DMA((N_DEV,)),
        ],
        compiler_params=pltpu.CompilerParams(collective_id=0),
    )(x)
